# baseline (speedup 1.0000x reference)
import functools

import jax
import jax.numpy as jnp
from jax import lax
from jax.experimental import pallas as pl
from jax.experimental.pallas import tpu as pltpu
from jax.experimental.pallas import tpu_sc as plsc

NC = 2
NS = 16
CH = 128
W = 64
EW = 128
HI = jax.lax.Precision.HIGHEST


def _sc_aggregate(src, dst, h, ef32):
    e = src.shape[0]
    n, dh = h.shape
    nv = e // CH
    tpb = -(-nv // (NS * NC))
    tpb = -(-tpb // 8) * 8
    nvpad = tpb * NS * NC
    rows_per_sub = -(-n // (8 * NS)) * 8
    npad = rows_per_sub * NS

    mesh = plsc.VectorSubcoreMesh(core_axis_name="c", subcore_axis_name="s",
                                  num_cores=NC)

    @functools.partial(
        pl.kernel,
        mesh=mesh,
        out_type=jax.ShapeDtypeStruct((NC, npad, dh), jnp.float32),
        scratch_types=[
            pltpu.VMEM((W, CH), jnp.int32),
            pltpu.VMEM((W, CH), jnp.int32),
            pltpu.VMEM((CH, dh), jnp.float32),
            pltpu.VMEM((CH, dh), jnp.float32),
            pltpu.VMEM_SHARED((npad, dh), jnp.float32),
            pltpu.SemaphoreType.DMA,
            pltpu.SemaphoreType.DMA,
        ],
    )
    def agg_h(src_hbm, dst_hbm, h_hbm, zh_hbm, out_hbm,
              srcw, dstw, r0, r1, acc_sh, s0, s1):
        s = lax.axis_index("s")
        c = lax.axis_index("c")
        zbase = s * rows_per_sub
        pltpu.sync_copy(zh_hbm, acc_sh.at[pl.ds(zbase, rows_per_sub)])
        plsc.subcore_barrier()

        start = (s * NC + c) * tpb
        nch = jnp.clip(nv - start, 0, tpb)
        nwin = -(-nch // W)

        def fire(g, rbuf, sem):
            pltpu.async_copy(h_hbm.at[srcw.at[g]], rbuf, sem)

        def wait(g, rbuf, sem):
            pltpu.make_async_copy(h_hbm.at[srcw.at[g]], rbuf, sem).wait()

        def scat(g, rbuf):
            pltpu.sync_copy(rbuf, acc_sh.at[dstw.at[g]], add=True)

        def window(w, carry):
            wbase = start + w * W
            nc = jnp.minimum(W, nch - w * W)
            last = nc - 1
            pltpu.sync_copy(src_hbm.at[pl.ds(wbase, W)], srcw)
            pltpu.sync_copy(dst_hbm.at[pl.ds(wbase, W)], dstw)
            fire(0, r0, s0)
            fire(jnp.minimum(1, last), r1, s1)

            def pair(p, c):
                g0 = 2 * p
                g1 = g0 + 1
                wait(g0, r0, s0)
                scat(g0, r0)
                fire(jnp.minimum(g0 + 2, last), r0, s0)
                wait(g1, r1, s1)
                scat(g1, r1)
                fire(jnp.minimum(g1 + 2, last), r1, s1)
                return c

            lax.fori_loop(0, nc // 2, pair, 0)
            wait(last, r0, s0)

            @pl.when(nc % 2 == 1)
            def _():
                scat(last, r0)

            wait(last, r1, s1)
            return carry

        lax.fori_loop(0, nwin, window, 0)
        plsc.subcore_barrier()
        pltpu.sync_copy(acc_sh.at[pl.ds(zbase, rows_per_sub)],
                        out_hbm.at[c, pl.ds(zbase, rows_per_sub)])

    @functools.partial(
        pl.kernel,
        mesh=mesh,
        out_type=jax.ShapeDtypeStruct((NC, npad, EW), jnp.float32),
        scratch_types=[
            pltpu.VMEM((W, CH), jnp.int32),
            pltpu.VMEM((CH, EW), jnp.float32),
            pltpu.VMEM((CH, EW), jnp.float32),
            pltpu.VMEM_SHARED((npad, EW), jnp.float32),
            pltpu.SemaphoreType.DMA,
            pltpu.SemaphoreType.DMA,
        ],
    )
    def agg_e(dst_hbm, ef_hbm, ze_hbm, out_hbm,
              dstw, e0, e1, acc_sh, s0, s1):
        s = lax.axis_index("s")
        c = lax.axis_index("c")
        zbase = s * rows_per_sub
        pltpu.sync_copy(ze_hbm, acc_sh.at[pl.ds(zbase, rows_per_sub)])
        plsc.subcore_barrier()

        start = (s * NC + c) * tpb
        nch = jnp.clip(nv - start, 0, tpb)
        nwin = -(-nch // W)

        def window(w, carry):
            wbase = start + w * W
            nc = jnp.minimum(W, nch - w * W)
            last = nc - 1
            pltpu.sync_copy(dst_hbm.at[pl.ds(wbase, W)], dstw)

            def fire(g, ebuf, sem):
                pltpu.async_copy(ef_hbm.at[pl.ds((wbase + g) * CH, CH)],
                                 ebuf, sem)

            def wait(g, ebuf, sem):
                pltpu.make_async_copy(
                    ef_hbm.at[pl.ds((wbase + g) * CH, CH)], ebuf, sem).wait()

            def scat(g, ebuf):
                pltpu.sync_copy(ebuf, acc_sh.at[dstw.at[g]], add=True)

            fire(0, e0, s0)
            fire(jnp.minimum(1, last), e1, s1)

            def pair(p, c):
                g0 = 2 * p
                g1 = g0 + 1
                wait(g0, e0, s0)
                scat(g0, e0)
                fire(jnp.minimum(g0 + 2, last), e0, s0)
                wait(g1, e1, s1)
                scat(g1, e1)
                fire(jnp.minimum(g1 + 2, last), e1, s1)
                return c

            lax.fori_loop(0, nc // 2, pair, 0)
            wait(last, e0, s0)

            @pl.when(nc % 2 == 1)
            def _():
                scat(last, e0)

            wait(last, e1, s1)
            return carry

        lax.fori_loop(0, nwin, window, 0)
        plsc.subcore_barrier()
        pltpu.sync_copy(acc_sh.at[pl.ds(zbase, rows_per_sub)],
                        out_hbm.at[c, pl.ds(zbase, rows_per_sub)])

    pad = nvpad - nv
    src2d = jnp.concatenate(
        [src.reshape(nv, CH), jnp.zeros((pad, CH), jnp.int32)])
    dst2d = jnp.concatenate(
        [dst.reshape(nv, CH), jnp.zeros((pad, CH), jnp.int32)])
    zh = jnp.zeros((rows_per_sub, dh), jnp.float32)
    ze = jnp.zeros((rows_per_sub, EW), jnp.float32)
    acc_h = agg_h(src2d, dst2d, h, zh)
    acc_e = agg_e(dst2d, ef32, ze)
    return acc_h, acc_e


def _final_body(nf_ref, h_ref, acch_ref, acce_ref, w1_ref, w2_ref, out_ref, *,
                dh, de):
    nf = nf_ref[...]
    acce = acce_ref[0] + acce_ref[1]
    acch = acch_ref[0] + acch_ref[1]
    efsum = acce[:, 0:de]
    deg = acce[:, de:de + 1]
    f32 = jnp.float32
    x = jnp.dot(nf, w1_ref[0:dh], precision=HI, preferred_element_type=f32)
    x += jnp.dot(deg * nf, w1_ref[dh:2 * dh], precision=HI,
                 preferred_element_type=f32)
    x += jnp.dot(efsum, w1_ref[2 * dh:2 * dh + de], precision=HI,
                 preferred_element_type=f32)
    x += jnp.dot(acch, w1_ref[2 * dh + de:], precision=HI,
                 preferred_element_type=f32)
    hn = jnp.dot(jnp.maximum(x, 0.0), w2_ref[...], precision=HI,
                 preferred_element_type=f32)
    out_ref[...] = jnp.where(deg > 0, hn, h_ref[...])


def _final(nf, h, acc_h, acc_e, W1, W2):
    n, dh = nf.shape
    in_dim, hid = W1.shape
    de = in_dim - 3 * dh
    nb = 1000
    return pl.pallas_call(
        functools.partial(_final_body, dh=dh, de=de),
        grid=(n // nb,),
        in_specs=[
            pl.BlockSpec((nb, dh), lambda i: (i, 0)),
            pl.BlockSpec((nb, dh), lambda i: (i, 0)),
            pl.BlockSpec((NC, nb, dh), lambda i: (0, i, 0)),
            pl.BlockSpec((NC, nb, EW), lambda i: (0, i, 0)),
            pl.BlockSpec((in_dim, hid), lambda i: (0, 0)),
            pl.BlockSpec((hid, hid), lambda i: (0, 0)),
        ],
        out_specs=pl.BlockSpec((nb, hid), lambda i: (i, 0)),
        out_shape=jax.ShapeDtypeStruct((n, hid), jnp.float32),
    )(nf, h, acc_h, acc_e, W1, W2)


def kernel(edge_index, h, nf, ef, W1, W2):
    src = edge_index[0].astype(jnp.int32)
    dst = edge_index[1].astype(jnp.int32)
    e, de = ef.shape
    ef32 = jnp.concatenate(
        [ef, jnp.ones((e, 1), jnp.float32),
         jnp.zeros((e, EW - de - 1), jnp.float32)], axis=1)
    acc_h, acc_e = _sc_aggregate(src, dst, h, ef32)
    return _final(nf, h, acc_h, acc_e, W1, W2)


# speedup vs baseline: 1.0021x; 1.0021x over previous
import functools

import jax
import jax.numpy as jnp
from jax import lax
from jax.experimental import pallas as pl
from jax.experimental.pallas import tpu as pltpu
from jax.experimental.pallas import tpu_sc as plsc

NC = 2
NS = 16
CH = 128
W = 48
EW = 128
HI = jax.lax.Precision.HIGHEST


def _sc_aggregate(src, dst, h, ef32):
    e = src.shape[0]
    n, dh = h.shape
    nv = e // CH
    tpb = -(-nv // (NS * NC))
    tpb = -(-tpb // 8) * 8
    nvpad = tpb * NS * NC
    rows_per_sub = -(-n // (8 * NS)) * 8
    npad = rows_per_sub * NS

    mesh = plsc.VectorSubcoreMesh(core_axis_name="c", subcore_axis_name="s",
                                  num_cores=NC)

    @functools.partial(
        pl.kernel,
        mesh=mesh,
        out_type=jax.ShapeDtypeStruct((NC, npad, dh), jnp.float32),
        scratch_types=[
            pltpu.VMEM((W, CH), jnp.int32),
            pltpu.VMEM((W, CH), jnp.int32),
            pltpu.VMEM((CH, dh), jnp.float32),
            pltpu.VMEM((CH, dh), jnp.float32),
            pltpu.VMEM_SHARED((npad, dh), jnp.float32),
            pltpu.SemaphoreType.DMA,
            pltpu.SemaphoreType.DMA,
        ],
    )
    def agg_h(src_hbm, dst_hbm, h_hbm, zh_hbm, out_hbm,
              srcw, dstw, r0, r1, acc_sh, s0, s1):
        s = lax.axis_index("s")
        c = lax.axis_index("c")
        zbase = s * rows_per_sub
        pltpu.sync_copy(zh_hbm, acc_sh.at[pl.ds(zbase, rows_per_sub)])
        plsc.subcore_barrier()

        start = (s * NC + c) * tpb
        nch = jnp.clip(nv - start, 0, tpb)
        nwin = -(-nch // W)

        def fire(g, rbuf, sem):
            pltpu.async_copy(h_hbm.at[srcw.at[g]], rbuf, sem)

        def wait(g, rbuf, sem):
            pltpu.make_async_copy(h_hbm.at[srcw.at[g]], rbuf, sem).wait()

        def scat(g, rbuf):
            pltpu.sync_copy(rbuf, acc_sh.at[dstw.at[g]], add=True)

        def window(w, carry):
            wbase = start + w * W
            nc = jnp.minimum(W, nch - w * W)
            last = nc - 1
            pltpu.sync_copy(src_hbm.at[pl.ds(wbase, W)], srcw)
            pltpu.sync_copy(dst_hbm.at[pl.ds(wbase, W)], dstw)
            fire(0, r0, s0)
            fire(jnp.minimum(1, last), r1, s1)

            def pair(p, c):
                g0 = 2 * p
                g1 = g0 + 1
                wait(g0, r0, s0)
                scat(g0, r0)
                fire(jnp.minimum(g0 + 2, last), r0, s0)
                wait(g1, r1, s1)
                scat(g1, r1)
                fire(jnp.minimum(g1 + 2, last), r1, s1)
                return c

            lax.fori_loop(0, nc // 2, pair, 0)
            wait(last, r0, s0)

            @pl.when(nc % 2 == 1)
            def _():
                scat(last, r0)

            wait(last, r1, s1)
            return carry

        lax.fori_loop(0, nwin, window, 0)
        plsc.subcore_barrier()
        pltpu.sync_copy(acc_sh.at[pl.ds(zbase, rows_per_sub)],
                        out_hbm.at[c, pl.ds(zbase, rows_per_sub)])

    @functools.partial(
        pl.kernel,
        mesh=mesh,
        out_type=jax.ShapeDtypeStruct((NC, npad, EW), jnp.float32),
        scratch_types=[
            pltpu.VMEM((W, CH), jnp.int32),
            pltpu.VMEM((CH, EW), jnp.float32),
            pltpu.VMEM((CH, EW), jnp.float32),
            pltpu.VMEM_SHARED((npad, EW), jnp.float32),
            pltpu.SemaphoreType.DMA,
            pltpu.SemaphoreType.DMA,
        ],
    )
    def agg_e(dst_hbm, ef_hbm, ze_hbm, out_hbm,
              dstw, e0, e1, acc_sh, s0, s1):
        s = lax.axis_index("s")
        c = lax.axis_index("c")
        zbase = s * rows_per_sub
        pltpu.sync_copy(ze_hbm, acc_sh.at[pl.ds(zbase, rows_per_sub)])
        plsc.subcore_barrier()

        start = (s * NC + c) * tpb
        nch = jnp.clip(nv - start, 0, tpb)
        nwin = -(-nch // W)

        def window(w, carry):
            wbase = start + w * W
            nc = jnp.minimum(W, nch - w * W)
            last = nc - 1
            pltpu.sync_copy(dst_hbm.at[pl.ds(wbase, W)], dstw)

            def fire(g, ebuf, sem):
                pltpu.async_copy(ef_hbm.at[pl.ds((wbase + g) * CH, CH)],
                                 ebuf, sem)

            def wait(g, ebuf, sem):
                pltpu.make_async_copy(
                    ef_hbm.at[pl.ds((wbase + g) * CH, CH)], ebuf, sem).wait()

            def scat(g, ebuf):
                pltpu.sync_copy(ebuf, acc_sh.at[dstw.at[g]], add=True)

            fire(0, e0, s0)
            fire(jnp.minimum(1, last), e1, s1)

            def pair(p, c):
                g0 = 2 * p
                g1 = g0 + 1
                wait(g0, e0, s0)
                scat(g0, e0)
                fire(jnp.minimum(g0 + 2, last), e0, s0)
                wait(g1, e1, s1)
                scat(g1, e1)
                fire(jnp.minimum(g1 + 2, last), e1, s1)
                return c

            lax.fori_loop(0, nc // 2, pair, 0)
            wait(last, e0, s0)

            @pl.when(nc % 2 == 1)
            def _():
                scat(last, e0)

            wait(last, e1, s1)
            return carry

        lax.fori_loop(0, nwin, window, 0)
        plsc.subcore_barrier()
        pltpu.sync_copy(acc_sh.at[pl.ds(zbase, rows_per_sub)],
                        out_hbm.at[c, pl.ds(zbase, rows_per_sub)])

    pad = nvpad - nv
    src2d = jnp.concatenate(
        [src.reshape(nv, CH), jnp.zeros((pad, CH), jnp.int32)])
    dst2d = jnp.concatenate(
        [dst.reshape(nv, CH), jnp.zeros((pad, CH), jnp.int32)])
    zh = jnp.zeros((rows_per_sub, dh), jnp.float32)
    ze = jnp.zeros((rows_per_sub, EW), jnp.float32)
    acc_h = agg_h(src2d, dst2d, h, zh)
    acc_e = agg_e(dst2d, ef32, ze)
    return acc_h, acc_e


def _final_body(nf_ref, h_ref, acch_ref, acce_ref, w1_ref, w2_ref, out_ref, *,
                dh, de):
    nf = nf_ref[...]
    acce = acce_ref[0] + acce_ref[1]
    acch = acch_ref[0] + acch_ref[1]
    efsum = acce[:, 0:de]
    deg = acce[:, de:de + 1]
    f32 = jnp.float32
    x = jnp.dot(nf, w1_ref[0:dh], precision=HI, preferred_element_type=f32)
    x += jnp.dot(deg * nf, w1_ref[dh:2 * dh], precision=HI,
                 preferred_element_type=f32)
    x += jnp.dot(efsum, w1_ref[2 * dh:2 * dh + de], precision=HI,
                 preferred_element_type=f32)
    x += jnp.dot(acch, w1_ref[2 * dh + de:], precision=HI,
                 preferred_element_type=f32)
    hn = jnp.dot(jnp.maximum(x, 0.0), w2_ref[...], precision=HI,
                 preferred_element_type=f32)
    out_ref[...] = jnp.where(deg > 0, hn, h_ref[...])


def _final(nf, h, acc_h, acc_e, W1, W2):
    n, dh = nf.shape
    in_dim, hid = W1.shape
    de = in_dim - 3 * dh
    nb = 1000
    return pl.pallas_call(
        functools.partial(_final_body, dh=dh, de=de),
        grid=(n // nb,),
        in_specs=[
            pl.BlockSpec((nb, dh), lambda i: (i, 0)),
            pl.BlockSpec((nb, dh), lambda i: (i, 0)),
            pl.BlockSpec((NC, nb, dh), lambda i: (0, i, 0)),
            pl.BlockSpec((NC, nb, EW), lambda i: (0, i, 0)),
            pl.BlockSpec((in_dim, hid), lambda i: (0, 0)),
            pl.BlockSpec((hid, hid), lambda i: (0, 0)),
        ],
        out_specs=pl.BlockSpec((nb, hid), lambda i: (i, 0)),
        out_shape=jax.ShapeDtypeStruct((n, hid), jnp.float32),
    )(nf, h, acc_h, acc_e, W1, W2)


def kernel(edge_index, h, nf, ef, W1, W2):
    src = edge_index[0].astype(jnp.int32)
    dst = edge_index[1].astype(jnp.int32)
    e, de = ef.shape
    ef32 = jnp.concatenate(
        [ef, jnp.ones((e, 1), jnp.float32),
         jnp.zeros((e, EW - de - 1), jnp.float32)], axis=1)
    acc_h, acc_e = _sc_aggregate(src, dst, h, ef32)
    return _final(nf, h, acc_h, acc_e, W1, W2)
